# speedup vs baseline: 17.6146x; 17.6146x over previous
import jax
import jax.numpy as jnp
from jax.experimental import pallas as pl
from jax.experimental.pallas import tpu as pltpu

_B = 1024


def _body(x_ref, o_ref):
    x = x_ref[...]
    p0 = x[:, 0, :] * x[:, 1, :] * x[:, 2, :] * x[:, 3, :] * x[:, 4, :]
    p1 = x[:, 5, :] * x[:, 6, :] * x[:, 7, :] * x[:, 8, :] * x[:, 9, :]
    o_ref[...] = jnp.stack([p0, p1], axis=1)


def kernel(inputs):
    n, r, d = inputs.shape
    grid = (n // _B,)
    return pl.pallas_call(
        _body,
        grid=grid,
        in_specs=[pl.BlockSpec((_B, r, d), lambda i: (i, 0, 0))],
        out_specs=pl.BlockSpec((_B, 2, d), lambda i: (i, 0, 0)),
        out_shape=jax.ShapeDtypeStruct((n, 2, d), inputs.dtype),
    )(inputs)
